# Initial kernel scaffold; baseline (speedup 1.0000x reference)
import jax
import jax.numpy as jnp
from jax.experimental import pallas as pl


def kernel(x, base_weight, spline_weight):
    raise NotImplementedError("write your pallas kernel here")


# speedup vs baseline: 919.9110x; 919.9110x over previous
import jax
import jax.numpy as jnp
import numpy as np
from jax.experimental import pallas as pl
from jax.experimental.pallas import tpu as pltpu

GRID_SIZE = 5
SPLINE_ORDER = 3
GRID_MIN = -1.0
GRID_MAX = 1.0
H = (GRID_MAX - GRID_MIN) / GRID_SIZE
G0 = GRID_MIN - SPLINE_ORDER * H
NCOEF = GRID_SIZE + SPLINE_ORDER
KP1 = SPLINE_ORDER + 1


def _power_basis_matrix():
    M = np.array([[1.0]], dtype=np.float64)
    scalar = 1.0
    for k in range(2, SPLINE_ORDER + 2):
        t1 = np.pad(M, ((0, 1), (0, 0)))
        t3 = np.pad(M, ((1, 0), (0, 0)))
        t2 = np.zeros((k - 1, k))
        t4 = np.zeros((k - 1, k))
        for i in range(k - 1):
            t2[i, i] = i + 1
            t2[i, i + 1] = k - (i + 2)
            t4[i, i] = -1.0
            t4[i, i + 1] = 1.0
        M = t1 @ t2 + t3 @ t4
        scalar *= 1.0 / (k - 1)
    return (M * scalar).astype(np.float32)


_B = _power_basis_matrix()


def _lane_permutation(fout):
    q_blocks = fout // KP1
    p = np.zeros((KP1 * fout, fout), dtype=np.float32)
    for r in range(KP1):
        for m in range(fout):
            p[fout * r + m, KP1 * (m % q_blocks) + r] = 1.0
    return p


def _kan_body(x_ref, wbt_ref, w_ref, p_ref, o_ref):
    x = x_ref[...]
    sx = x * jax.lax.logistic(x)
    base = jnp.dot(sx, wbt_ref[...], preferred_element_type=jnp.float32)
    xn = jnp.clip(x, GRID_MIN, GRID_MAX)
    t = (xn - G0) / H
    fl = jnp.floor(t)
    u = t - fl
    cf = jnp.clip(fl.astype(jnp.int32), SPLINE_ORDER,
                  SPLINE_ORDER + GRID_SIZE - 1) - SPLINE_ORDER
    u2 = u * u
    u3 = u2 * u
    cols = [
        _B[0, j] + _B[1, j] * u + _B[2, j] * u2 + _B[3, j] * u3
        for j in range(KP1)
    ]
    L = jnp.concatenate(cols, axis=1)
    T = jnp.dot(L, w_ref[...], preferred_element_type=jnp.float32)
    fin = x.shape[1]
    t_blocks = [T[:, fin * c:fin * (c + 1)] for c in range(NCOEF)]
    u_blocks = []
    for r in range(KP1):
        acc = jnp.where(cf == 0, t_blocks[r], 0.0)
        for s in range(1, GRID_SIZE):
            acc = acc + jnp.where(cf == s, t_blocks[s + r], 0.0)
        u_blocks.append(acc)
    U = jnp.concatenate(u_blocks, axis=1)
    o_ref[...] = base + jnp.dot(U, p_ref[...],
                                preferred_element_type=jnp.float32)


def _run(x, base_weight, spline_weight, interpret=False):
    n, fin = x.shape
    fout = base_weight.shape[0]
    cp = jnp.reshape(spline_weight, (fout, fin, NCOEF))
    cp_flat = jnp.transpose(cp, (0, 2, 1)).reshape(fin, NCOEF * fin)
    col_j = (jnp.arange(NCOEF * fin) % fin) // (fin // KP1)
    w = jnp.where(col_j[None, None, :] == jnp.arange(KP1)[:, None, None],
                  cp_flat[None, :, :], 0.0).reshape(KP1 * fin, NCOEF * fin)
    p = jnp.asarray(_lane_permutation(fout))
    wbt = base_weight.T

    bn = 256
    return pl.pallas_call(
        _kan_body,
        out_shape=jax.ShapeDtypeStruct((n, fout), x.dtype),
        grid=(n // bn,),
        in_specs=[
            pl.BlockSpec((bn, fin), lambda i: (i, 0)),
            pl.BlockSpec((fin, fout), lambda i: (0, 0)),
            pl.BlockSpec((KP1 * fin, NCOEF * fin), lambda i: (0, 0)),
            pl.BlockSpec((KP1 * fin, fout), lambda i: (0, 0)),
        ],
        out_specs=pl.BlockSpec((bn, fout), lambda i: (i, 0)),
        compiler_params=pltpu.CompilerParams(
            dimension_semantics=("parallel",),
        ),
        name="matrix_kan_linear",
        interpret=interpret,
    )(x, wbt, w, p)


def kernel(x, base_weight, spline_weight):
    return _run(x, base_weight, spline_weight)
